# Initial kernel scaffold; baseline (speedup 1.0000x reference)
import functools

import jax
import jax.numpy as jnp
from jax import lax
from jax.experimental import pallas as pl
from jax.experimental.pallas import tpu as pltpu
from jax.experimental.pallas import tpu_sc as plsc

N = 10000
R = 10240
TRASH = N
E = 320000
NTILES = 32
CH = 128
NCHUNK = 80
EPAD = NTILES * NCHUNK * CH
NB = 4
RPT = R // 16
NGRID = 8
BLK = R // NGRID


def _make_agg(d):
  mesh = plsc.VectorSubcoreMesh(core_axis_name="c", subcore_axis_name="s")

  def body(table, srcs, dsts, zeros, out, src_v, dst_v, bufs, acc, gsem, ssem):
    c = lax.axis_index("c")
    s = lax.axis_index("s")
    wid = s * 2 + c
    pltpu.sync_copy(srcs.at[wid], src_v)
    pltpu.sync_copy(dsts.at[wid], dst_v)
    pltpu.sync_copy(zeros, acc.at[pl.ds(s * RPT, RPT)])
    plsc.subcore_barrier()

    pltpu.sync_copy(zeros.at[pl.ds(0, CH)], bufs.at[2])
    pltpu.sync_copy(zeros.at[pl.ds(0, CH)], bufs.at[3])
    pltpu.async_copy(table.at[src_v.at[0]], bufs.at[0], gsem.at[0])
    pltpu.async_copy(table.at[src_v.at[1]], bufs.at[1], gsem.at[1])
    pltpu.async_copy(bufs.at[2], acc.at[dst_v.at[0]], ssem.at[2], add=True)
    pltpu.async_copy(bufs.at[3], acc.at[dst_v.at[1]], ssem.at[3], add=True)

    def group(g, carry):
      j0 = g * NB
      for b in range(NB):
        j = j0 + b
        b2 = (b + 2) % NB
        pltpu.make_async_copy(table.at[src_v.at[j]], bufs.at[b],
                              gsem.at[b]).wait()
        pltpu.async_copy(bufs.at[b], acc.at[dst_v.at[j]], ssem.at[b], add=True)
        pltpu.make_async_copy(bufs.at[b2], acc.at[dst_v.at[0]],
                              ssem.at[b2]).wait()
        pltpu.async_copy(table.at[src_v.at[j + 2]], bufs.at[b2], gsem.at[b2])
      return carry

    lax.fori_loop(0, NCHUNK // NB, group, 0)

    for b in (0, 1):
      pltpu.make_async_copy(table.at[src_v.at[0]], bufs.at[b],
                            gsem.at[b]).wait()
    for b in (2, 3):
      pltpu.make_async_copy(bufs.at[b], acc.at[dst_v.at[0]],
                            ssem.at[b]).wait()
    plsc.subcore_barrier()
    pltpu.sync_copy(acc.at[pl.ds(s * RPT, RPT)],
                    out.at[c, pl.ds(s * RPT, RPT)])

  return pl.kernel(
      body,
      out_type=jax.ShapeDtypeStruct((2, R, d), jnp.float32),
      mesh=mesh,
      scratch_types=[
          pltpu.VMEM((NCHUNK + NB, CH), jnp.int32),
          pltpu.VMEM((NCHUNK, CH), jnp.int32),
          pltpu.VMEM((NB, CH, d), jnp.float32),
          pltpu.VMEM_SHARED((R, d), jnp.float32),
          pltpu.SemaphoreType.DMA((NB,)),
          pltpu.SemaphoreType.DMA((NB,)),
      ],
  )


_agg = {d: _make_agg(d) for d in (16, 32, 64, 128)}


def _node(d):
  return pl.BlockSpec((BLK, d), lambda i: (i, 0))


def _part(d):
  return pl.BlockSpec((2, BLK, d), lambda i: (0, i, 0))


def _whole(shape):
  nd = len(shape)
  return pl.BlockSpec(shape, lambda i: (0,) * nd)


def _tc(body, in_specs, out_specs, out_shape):
  return pl.pallas_call(body, grid=(NGRID,), in_specs=in_specs,
                        out_specs=out_specs, out_shape=out_shape)


def _relu(v):
  return jnp.maximum(v, 0.0)


def _t1(xr, p0, dg, wl, bl, wr, h1a, h1b, ivd):
  iv = 1.0 / jnp.maximum(dg[0] + dg[1], 1.0)
  ivd[...] = iv
  a = (p0[0] + p0[1]) * iv[:, 0:1]
  h = _relu(jnp.dot(a, wl[...]) + jnp.dot(xr[...], wr[...]) + bl[...])
  h1a[...] = h[:, :128]
  h1b[...] = h[:, 128:]


def _t2(h1a, h1b, pa, pb, ivd, wla, wlb, bl, wra, wrb, wn, h2a, h2b, g2):
  iv = ivd[:, 0:1]
  aa = (pa[0] + pa[1]) * iv
  ab = (pb[0] + pb[1]) * iv
  h = _relu(jnp.dot(aa, wla[...]) + jnp.dot(ab, wlb[...]) +
            jnp.dot(h1a[...], wra[...]) + jnp.dot(h1b[...], wrb[...]) +
            bl[...])
  h2a[...] = h[:, :128]
  h2b[...] = h[:, 128:]
  g2[...] = jnp.dot(h, wn[...])


def _t3(h2a, h2b, p2, ivd, bl, wra, wrb, h3):
  a = (p2[0] + p2[1]) * ivd[:, 0:1]
  h3[...] = _relu(a + jnp.dot(h2a[...], wra[...]) +
                  jnp.dot(h2b[...], wrb[...]) + bl[...])


def _t4(hp, pp, ivd, wl, bl, wr, wn, hn, gn):
  a = (pp[0] + pp[1]) * ivd[:, 0:1]
  h = _relu(jnp.dot(a, wl[...]) + jnp.dot(hp[...], wr[...]) + bl[...])
  hn[...] = h
  gn[...] = jnp.dot(h, wn[...])


def _t5(hp, pp, ivd, bl, wr, hn):
  a = (pp[0] + pp[1]) * ivd[:, 0:1]
  hn[...] = _relu(a + jnp.dot(hp[...], wr[...]) + bl[...])


def _t8(h7, p7, ivd, wl, bl, wr, wreg8, breg, y8):
  a = (p7[0] + p7[1]) * ivd[:, 0:1]
  h = _relu(jnp.dot(a, wl[...]) + jnp.dot(h7[...], wr[...]) + bl[...])
  y8[...] = jnp.dot(h, wreg8[...]) + breg[...]


def kernel(x, edge_index,
           Wl0, bl0, Wr0, Wl1, bl1, Wr1, Wl2, bl2, Wr2, Wl3, bl3, Wr3,
           Wl4, bl4, Wr4, Wl5, bl5, Wr5, Wl6, bl6, Wr6, Wl7, bl7, Wr7,
           Wreg, breg):
  f32 = jnp.float32
  src = edge_index[0]
  dst = edge_index[1]
  pad = EPAD - E
  srcp = jnp.concatenate([src, jnp.zeros((pad,), jnp.int32)])
  srcp = srcp.reshape(NTILES, NCHUNK, CH)
  srcp = jnp.concatenate([srcp, jnp.zeros((NTILES, NB, CH), jnp.int32)],
                         axis=1)
  dstp = jnp.concatenate([dst, jnp.full((pad,), TRASH, jnp.int32)])
  dstp = dstp.reshape(NTILES, NCHUNK, CH)

  xp = jnp.zeros((R, 128), f32).at[:N].set(x)
  ones16 = jnp.ones((R, 16), f32)
  zer = {d: jnp.zeros((RPT, d), f32) for d in (16, 32, 64, 128)}
  b = {i: v.reshape(1, -1) for i, v in
       enumerate([bl0, bl1, bl2, bl3, bl4, bl5, bl6, bl7])}
  wreg8 = jnp.tile(Wreg, (1, 8))
  breg8 = jnp.broadcast_to(breg, (8,)).reshape(1, 8)

  D = _agg[16](ones16, srcp, dstp, zer[16])
  P0 = _agg[128](xp, srcp, dstp, zer[128])

  h1a, h1b, ivd = _tc(
      _t1,
      [_node(128), _part(128), _part(16), _whole((128, 256)), _whole((1, 256)),
       _whole((128, 256))],
      [_node(128), _node(128), _node(16)],
      [jax.ShapeDtypeStruct((R, 128), f32)] * 2 +
      [jax.ShapeDtypeStruct((R, 16), f32)],
  )(xp, P0, D, Wl0, b[0], Wr0)

  Pa = _agg[128](h1a, srcp, dstp, zer[128])
  Pb = _agg[128](h1b, srcp, dstp, zer[128])
  h2a, h2b, g2 = _tc(
      _t2,
      [_node(128), _node(128), _part(128), _part(128), _node(16),
       _whole((128, 256)), _whole((128, 256)), _whole((1, 256)),
       _whole((128, 256)), _whole((128, 256)), _whole((256, 128))],
      [_node(128), _node(128), _node(128)],
      [jax.ShapeDtypeStruct((R, 128), f32)] * 3,
  )(h1a, h1b, Pa, Pb, ivd, Wl1[:128], Wl1[128:], b[1], Wr1[:128], Wr1[128:],
    Wl2)

  P2 = _agg[128](g2, srcp, dstp, zer[128])
  h3 = _tc(
      _t3,
      [_node(128), _node(128), _part(128), _node(16), _whole((1, 128)),
       _whole((128, 128)), _whole((128, 128))],
      _node(128),
      jax.ShapeDtypeStruct((R, 128), f32),
  )(h2a, h2b, P2, ivd, b[2], Wr2[:128], Wr2[128:])

  P3 = _agg[128](h3, srcp, dstp, zer[128])
  h4, g4 = _tc(
      _t4,
      [_node(128), _part(128), _node(16), _whole((128, 128)),
       _whole((1, 128)), _whole((128, 128)), _whole((128, 64))],
      [_node(128), _node(64)],
      [jax.ShapeDtypeStruct((R, 128), f32), jax.ShapeDtypeStruct((R, 64), f32)],
  )(h3, P3, ivd, Wl3, b[3], Wr3, Wl4)

  P4 = _agg[64](g4, srcp, dstp, zer[64])
  h5 = _tc(
      _t5,
      [_node(128), _part(64), _node(16), _whole((1, 64)), _whole((128, 64))],
      _node(64),
      jax.ShapeDtypeStruct((R, 64), f32),
  )(h4, P4, ivd, b[4], Wr4)

  P5 = _agg[64](h5, srcp, dstp, zer[64])
  h6, g6 = _tc(
      _t4,
      [_node(64), _part(64), _node(16), _whole((64, 64)), _whole((1, 64)),
       _whole((64, 64)), _whole((64, 32))],
      [_node(64), _node(32)],
      [jax.ShapeDtypeStruct((R, 64), f32), jax.ShapeDtypeStruct((R, 32), f32)],
  )(h5, P5, ivd, Wl5, b[5], Wr5, Wl6)

  P6 = _agg[32](g6, srcp, dstp, zer[32])
  h7 = _tc(
      _t5,
      [_node(64), _part(32), _node(16), _whole((1, 32)), _whole((64, 32))],
      _node(32),
      jax.ShapeDtypeStruct((R, 32), f32),
  )(h6, P6, ivd, b[6], Wr6)

  P7 = _agg[32](h7, srcp, dstp, zer[32])
  y8 = _tc(
      _t8,
      [_node(32), _part(32), _node(16), _whole((32, 32)), _whole((1, 32)),
       _whole((32, 32)), _whole((32, 8)), _whole((1, 8))],
      _node(8),
      jax.ShapeDtypeStruct((R, 8), f32),
  )(h7, P7, ivd, Wl7, b[7], Wr7, wreg8, breg8)

  return y8[:N, :1]


# speedup vs baseline: 4.1548x; 4.1548x over previous
import jax
import jax.numpy as jnp
from jax import lax
from jax.experimental import pallas as pl
from jax.experimental.pallas import tpu as pltpu
from jax.experimental.pallas import tpu_sc as plsc

N = 10000
R = 10240
TRASH = N
E = 320000
NSUB = 16
CH = 128
NCHUNK = 160
EPAD = NSUB * NCHUNK * CH
NB = 4
RPT = R // NSUB
NGRID = 8
BLK = R // NGRID


def _make_agg(dh):
  mesh = plsc.VectorSubcoreMesh(core_axis_name="c", subcore_axis_name="s")

  def body(table, srcs, dsts, zeros, out, src_v, dst_v, bufs, acc, gsem, ssem):
    c = lax.axis_index("c")
    s = lax.axis_index("s")
    pltpu.sync_copy(srcs.at[c * NSUB + s], src_v)
    pltpu.sync_copy(dsts.at[s], dst_v)
    pltpu.sync_copy(zeros, acc.at[pl.ds(s * RPT, RPT)])
    plsc.subcore_barrier()

    pltpu.sync_copy(zeros.at[pl.ds(0, CH)], bufs.at[2])
    pltpu.sync_copy(zeros.at[pl.ds(0, CH)], bufs.at[3])
    pltpu.async_copy(table.at[src_v.at[0]], bufs.at[0], gsem.at[0])
    pltpu.async_copy(table.at[src_v.at[1]], bufs.at[1], gsem.at[1])
    pltpu.async_copy(bufs.at[2], acc.at[dst_v.at[0]], ssem.at[2], add=True)
    pltpu.async_copy(bufs.at[3], acc.at[dst_v.at[1]], ssem.at[3], add=True)

    def group(g, carry):
      j0 = g * NB
      for b in range(NB):
        j = j0 + b
        b2 = (b + 2) % NB
        pltpu.make_async_copy(table.at[src_v.at[j]], bufs.at[b],
                              gsem.at[b]).wait()
        pltpu.async_copy(bufs.at[b], acc.at[dst_v.at[j]], ssem.at[b], add=True)
        pltpu.make_async_copy(bufs.at[b2], acc.at[dst_v.at[0]],
                              ssem.at[b2]).wait()
        pltpu.async_copy(table.at[src_v.at[j + 2]], bufs.at[b2], gsem.at[b2])
      return carry

    lax.fori_loop(0, NCHUNK // NB, group, 0)

    for b in (0, 1):
      pltpu.make_async_copy(table.at[src_v.at[0]], bufs.at[b],
                            gsem.at[b]).wait()
    for b in (2, 3):
      pltpu.make_async_copy(bufs.at[b], acc.at[dst_v.at[0]],
                            ssem.at[b]).wait()
    plsc.subcore_barrier()
    pltpu.sync_copy(acc.at[pl.ds(s * RPT, RPT)],
                    out.at[c, pl.ds(s * RPT, RPT)])

  return pl.kernel(
      body,
      out_type=jax.ShapeDtypeStruct((2, R, dh), jnp.float32),
      mesh=mesh,
      compiler_params=pltpu.CompilerParams(use_tc_tiling_on_sc=False),
      scratch_types=[
          pltpu.VMEM((NCHUNK + NB, CH), jnp.int32),
          pltpu.VMEM((NCHUNK, CH), jnp.int32),
          pltpu.VMEM((NB, CH, dh), jnp.float32),
          pltpu.VMEM_SHARED((R, dh), jnp.float32),
          pltpu.SemaphoreType.DMA((NB,)),
          pltpu.SemaphoreType.DMA((NB,)),
      ],
  )


_agg = {dh: _make_agg(dh) for dh in (16, 32, 64)}


def _node(d):
  return pl.BlockSpec((BLK, d), lambda i: (i, 0))


def _half(dh):
  return pl.BlockSpec((2, BLK, dh), lambda i: (0, i, 0))


def _whole(shape):
  nd = len(shape)
  return pl.BlockSpec(shape, lambda i: (0,) * nd)


def _tc(body, in_specs, out_specs, out_shape):
  return pl.pallas_call(body, grid=(NGRID,), in_specs=in_specs,
                        out_specs=out_specs, out_shape=out_shape)


def _relu(v):
  return jnp.maximum(v, 0.0)


def _mmh(parts, w):
  o = 0
  acc = None
  for p in parts:
    dh = p.shape[1]
    t = jnp.dot(p, w[o:o + dh])
    acc = t if acc is None else acc + t
    o += dh
  return acc


def _wr_half(ref, h):
  dh = h.shape[1] // 2
  ref[0] = h[:, :dh]
  ref[1] = h[:, dh:]


def _t1(xs, p0, dg, wl, bl, wr, h1a, h1b, ivd):
  iv = 1.0 / jnp.maximum(dg[0][:, 0:1], 1.0)
  ivd[...] = jnp.broadcast_to(iv, ivd.shape)
  h = _relu(_mmh([p0[0] * iv, p0[1] * iv], wl[...]) +
            _mmh([xs[0], xs[1]], wr[...]) + bl[...])
  h1a[0] = h[:, 0:64]
  h1a[1] = h[:, 64:128]
  h1b[0] = h[:, 128:192]
  h1b[1] = h[:, 192:256]


def _t2(h1a, h1b, pa, pb, ivd, wl, bl, wr, wn, h2a, h2b, g2):
  iv = ivd[:, 0:1]
  agg = [pa[0] * iv, pa[1] * iv, pb[0] * iv, pb[1] * iv]
  h = _relu(_mmh(agg, wl[...]) +
            _mmh([h1a[0], h1a[1], h1b[0], h1b[1]], wr[...]) + bl[...])
  h2a[0] = h[:, 0:64]
  h2a[1] = h[:, 64:128]
  h2b[0] = h[:, 128:192]
  h2b[1] = h[:, 192:256]
  _wr_half(g2, jnp.dot(h, wn[...]))


def _t3(h2a, h2b, p2, ivd, bl, wr, h3):
  iv = ivd[:, 0:1]
  m = _mmh([h2a[0], h2a[1], h2b[0], h2b[1]], wr[...]) + bl[...]
  h3[0] = _relu(p2[0] * iv + m[:, :64])
  h3[1] = _relu(p2[1] * iv + m[:, 64:])


def _t4(hp, pp, ivd, wl, bl, wr, wn, hn, gn):
  iv = ivd[:, 0:1]
  h = _relu(_mmh([pp[0] * iv, pp[1] * iv], wl[...]) +
            _mmh([hp[0], hp[1]], wr[...]) + bl[...])
  _wr_half(hn, h)
  _wr_half(gn, jnp.dot(h, wn[...]))


def _t5(hp, pp, ivd, bl, wr, hn):
  iv = ivd[:, 0:1]
  m = _mmh([hp[0], hp[1]], wr[...]) + bl[...]
  dh = m.shape[1] // 2
  hn[0] = _relu(pp[0] * iv + m[:, :dh])
  hn[1] = _relu(pp[1] * iv + m[:, dh:])


def _t8(h7, p7, ivd, wl, bl, wr, wreg8, breg, y8):
  iv = ivd[:, 0:1]
  h = _relu(_mmh([p7[0] * iv, p7[1] * iv], wl[...]) +
            _mmh([h7[0], h7[1]], wr[...]) + bl[...])
  y8[...] = jnp.dot(h, wreg8[...]) + breg[...]


def kernel(x, edge_index,
           Wl0, bl0, Wr0, Wl1, bl1, Wr1, Wl2, bl2, Wr2, Wl3, bl3, Wr3,
           Wl4, bl4, Wr4, Wl5, bl5, Wr5, Wl6, bl6, Wr6, Wl7, bl7, Wr7,
           Wreg, breg):
  f32 = jnp.float32
  src = edge_index[0]
  dst = edge_index[1]
  pad = EPAD - E
  srcp = jnp.concatenate([src, jnp.zeros((pad,), jnp.int32)])
  srcp = srcp.reshape(NSUB, NCHUNK, CH)
  srcp = jnp.concatenate([srcp, jnp.zeros((NSUB, NB, CH), jnp.int32)], axis=1)
  srcs = jnp.concatenate([srcp, srcp + R])
  dstp = jnp.concatenate([dst, jnp.full((pad,), TRASH, jnp.int32)])
  dstp = dstp.reshape(NSUB, NCHUNK, CH)

  xp = jnp.zeros((R, 128), f32).at[:N].set(x)
  xs = jnp.concatenate([xp[:, :64], xp[:, 64:]], axis=0)
  ones16 = jnp.ones((2 * R, 16), f32)
  zer = {d: jnp.zeros((RPT, d), f32) for d in (16, 32, 64)}
  b = {i: v.reshape(1, -1) for i, v in
       enumerate([bl0, bl1, bl2, bl3, bl4, bl5, bl6, bl7])}
  wreg8 = jnp.tile(Wreg, (1, 8))
  breg8 = jnp.broadcast_to(breg, (8,)).reshape(1, 8)

  def agg(dh, table):
    return _agg[dh](table.reshape(2 * R, dh), srcs, dstp, zer[dh])

  D = _agg[16](ones16, srcs, dstp, zer[16])
  P0 = agg(64, xs)

  h1a, h1b, ivd = _tc(
      _t1,
      [_half(64), _half(64), _half(16), _whole((128, 256)), _whole((1, 256)),
       _whole((128, 256))],
      [_half(64), _half(64), _node(16)],
      [jax.ShapeDtypeStruct((2, R, 64), f32)] * 2 +
      [jax.ShapeDtypeStruct((R, 16), f32)],
  )(xs.reshape(2, R, 64), P0, D, Wl0, b[0], Wr0)

  Pa = agg(64, h1a)
  Pb = agg(64, h1b)
  h2a, h2b, g2 = _tc(
      _t2,
      [_half(64), _half(64), _half(64), _half(64), _node(16),
       _whole((256, 256)), _whole((1, 256)), _whole((256, 256)),
       _whole((256, 128))],
      [_half(64), _half(64), _half(64)],
      [jax.ShapeDtypeStruct((2, R, 64), f32)] * 3,
  )(h1a, h1b, Pa, Pb, ivd, Wl1, b[1], Wr1, Wl2)

  P2 = agg(64, g2)
  h3 = _tc(
      _t3,
      [_half(64), _half(64), _half(64), _node(16), _whole((1, 128)),
       _whole((256, 128))],
      _half(64),
      jax.ShapeDtypeStruct((2, R, 64), f32),
  )(h2a, h2b, P2, ivd, b[2], Wr2)

  P3 = agg(64, h3)
  h4, g4 = _tc(
      _t4,
      [_half(64), _half(64), _node(16), _whole((128, 128)), _whole((1, 128)),
       _whole((128, 128)), _whole((128, 64))],
      [_half(64), _half(32)],
      [jax.ShapeDtypeStruct((2, R, 64), f32),
       jax.ShapeDtypeStruct((2, R, 32), f32)],
  )(h3, P3, ivd, Wl3, b[3], Wr3, Wl4)

  P4 = agg(32, g4)
  h5 = _tc(
      _t5,
      [_half(64), _half(32), _node(16), _whole((1, 64)), _whole((128, 64))],
      _half(32),
      jax.ShapeDtypeStruct((2, R, 32), f32),
  )(h4, P4, ivd, b[4], Wr4)

  P5 = agg(32, h5)
  h6, g6 = _tc(
      _t4,
      [_half(32), _half(32), _node(16), _whole((64, 64)), _whole((1, 64)),
       _whole((64, 64)), _whole((64, 32))],
      [_half(32), _half(16)],
      [jax.ShapeDtypeStruct((2, R, 32), f32),
       jax.ShapeDtypeStruct((2, R, 16), f32)],
  )(h5, P5, ivd, Wl5, b[5], Wr5, Wl6)

  P6 = agg(16, g6)
  h7 = _tc(
      _t5,
      [_half(32), _half(16), _node(16), _whole((1, 32)), _whole((64, 32))],
      _half(16),
      jax.ShapeDtypeStruct((2, R, 16), f32),
  )(h6, P6, ivd, b[6], Wr6)

  P7 = agg(16, h7)
  y8 = _tc(
      _t8,
      [_half(16), _half(16), _node(16), _whole((32, 32)), _whole((1, 32)),
       _whole((32, 32)), _whole((32, 8)), _whole((1, 8))],
      _node(8),
      jax.ShapeDtypeStruct((R, 8), f32),
  )(h7, P7, ivd, Wl7, b[7], Wr7, wreg8, breg8)

  return y8[:N, :1]
